# baseline (speedup 1.0000x reference)
import functools

import jax
import jax.numpy as jnp
from jax import lax
from jax.experimental import pallas as pl
from jax.experimental.pallas import tpu as pltpu
from jax.experimental.pallas import tpu_sc as plsc

DIM = 64
BATCH = 16384
HIST = 20
B = BATCH * HIST

NC, NS = 2, 16
NW = NC * NS

GRP = 128
G_PER_W = B // (NW * GRP)
K = 8


def _build():
    mesh = plsc.VectorSubcoreMesh(core_axis_name="c", subcore_axis_name="s")

    @functools.partial(
        pl.kernel,
        mesh=mesh,
        out_type=jax.ShapeDtypeStruct((B // GRP, GRP, DIM), jnp.float32),
        scratch_types=[
            pltpu.VMEM((G_PER_W, GRP), jnp.int32),
            pltpu.VMEM((K, GRP, DIM), jnp.float32),
            pltpu.SemaphoreType.DMA,
        ],
        compiler_params=pltpu.CompilerParams(use_tc_tiling_on_sc=False),
    )
    def gather_kernel(idx_hbm, table_hbm, out_hbm, idx_v, rows_v, sem):
        wid = lax.axis_index("s") * NC + lax.axis_index("c")
        base = wid * G_PER_W
        pltpu.sync_copy(idx_hbm.at[pl.ds(base, G_PER_W)], idx_v)

        def body(g, carry):
            g0 = g * K
            cps = [
                pltpu.async_copy(table_hbm.at[idx_v.at[g0 + b]], rows_v.at[b], sem)
                for b in range(K)
            ]
            for cp in cps:
                cp.wait()
            pltpu.sync_copy(rows_v, out_hbm.at[pl.ds(base + g0, K)])
            return carry

        lax.fori_loop(0, G_PER_W // K, body, 0)

    return gather_kernel


_gather = _build()


def kernel(indices, weight):
    idx = indices.astype(jnp.int32).reshape(B // GRP, GRP)
    out = _gather(idx, weight)
    return out.reshape(BATCH, HIST, DIM)


# speedup vs baseline: 1.0038x; 1.0038x over previous
import functools

import jax
import jax.numpy as jnp
from jax import lax
from jax.experimental import pallas as pl
from jax.experimental.pallas import tpu as pltpu
from jax.experimental.pallas import tpu_sc as plsc

DIM = 64
BATCH = 16384
HIST = 20
B = BATCH * HIST

NC, NS = 2, 16
NW = NC * NS

GRP = 128
S_PER_W = B // (NW * GRP)
K = 5
NPAIR = S_PER_W // (2 * K)


def _build():
    mesh = plsc.VectorSubcoreMesh(core_axis_name="c", subcore_axis_name="s")

    @functools.partial(
        pl.kernel,
        mesh=mesh,
        out_type=jax.ShapeDtypeStruct((B // GRP, GRP, DIM), jnp.float32),
        scratch_types=[
            pltpu.VMEM((S_PER_W, GRP), jnp.int32),
            pltpu.VMEM((2, K, GRP, DIM), jnp.float32),
            pltpu.SemaphoreType.DMA,
            pltpu.SemaphoreType.DMA,
            pltpu.SemaphoreType.DMA,
            pltpu.SemaphoreType.DMA,
        ],
        compiler_params=pltpu.CompilerParams(use_tc_tiling_on_sc=False),
    )
    def gather_kernel(idx_hbm, table_hbm, out_hbm, idx_v, rows_v, g0, g1, s0, s1):
        wid = lax.axis_index("s") * NC + lax.axis_index("c")
        base = wid * S_PER_W
        pltpu.sync_copy(idx_hbm.at[pl.ds(base, S_PER_W)], idx_v)
        gsem = (g0, g1)
        ssem = (s0, s1)

        def scat_wait(buf):
            pltpu.make_async_copy(
                rows_v.at[buf], out_hbm.at[pl.ds(base, K)], ssem[buf]
            ).wait()

        def body(i, carry):
            for buf in range(2):
                st = (2 * i + buf) * K

                @pl.when(i > 0)
                def _free_buf():
                    scat_wait(buf)

                cps = [
                    pltpu.async_copy(
                        table_hbm.at[idx_v.at[st + b]], rows_v.at[buf, b], gsem[buf]
                    )
                    for b in range(K)
                ]
                for cp in cps:
                    cp.wait()
                pltpu.async_copy(
                    rows_v.at[buf], out_hbm.at[pl.ds(base + st, K)], ssem[buf]
                )
            return carry

        lax.fori_loop(0, NPAIR, body, 0)
        scat_wait(0)
        scat_wait(1)

    return gather_kernel


_gather = _build()


def kernel(indices, weight):
    idx = indices.astype(jnp.int32).reshape(B // GRP, GRP)
    out = _gather(idx, weight)
    return out.reshape(BATCH, HIST, DIM)
